# baseline (speedup 1.0000x reference)
import functools

import jax
import jax.numpy as jnp
from jax import lax
from jax.experimental import pallas as pl
from jax.experimental.pallas import tpu as pltpu
from jax.experimental.pallas import tpu_sc as plsc

BATCH = 16384
DIM = 32
L = 16
NC, NS = 2, 16
NW = NC * NS
BPW = BATCH // NW
CHUNK = 128
NCHUNK = BPW // CHUNK

_mesh = plsc.VectorSubcoreMesh(core_axis_name="c", subcore_axis_name="s")


@functools.partial(
    pl.kernel,
    mesh=_mesh,
    out_type=jax.ShapeDtypeStruct((BATCH,), jnp.float32),
    compiler_params=pltpu.CompilerParams(
        needs_layout_passes=False, use_tc_tiling_on_sc=False),
    scratch_types=[
        pltpu.VMEM((NCHUNK, CHUNK), jnp.int32),
        pltpu.VMEM((NCHUNK, CHUNK), jnp.int32),
        pltpu.VMEM((BPW, DIM), jnp.float32),
        pltpu.VMEM((BPW, DIM), jnp.float32),
        pltpu.VMEM((BPW,), jnp.float32),
        pltpu.SemaphoreType.DMA,
    ],
)
def _mf_sc(x0_hbm, x1_hbm, u_hbm, v_hbm, out_hbm,
           idx0_v, idx1_v, urows_v, vrows_v, out_v, sem):
    wid = lax.axis_index("s") * NC + lax.axis_index("c")

    pltpu.sync_copy(x0_hbm.at[pl.ds(wid * NCHUNK, NCHUNK)], idx0_v)
    pltpu.sync_copy(x1_hbm.at[pl.ds(wid * NCHUNK, NCHUNK)], idx1_v)

    copies = []
    for j in range(NCHUNK):
        copies.append(pltpu.async_copy(
            u_hbm.at[idx0_v.at[j]], urows_v.at[pl.ds(j * CHUNK, CHUNK)], sem))
        copies.append(pltpu.async_copy(
            v_hbm.at[idx1_v.at[j]], vrows_v.at[pl.ds(j * CHUNK, CHUNK)], sem))
    for c in copies:
        c.wait()

    lane = lax.iota(jnp.int32, L)

    def body(g, carry):
        rid = g * L + lane
        acc = jnp.zeros((L,), jnp.float32)
        for d in range(DIM):
            dcol = jnp.full((L,), d, jnp.int32)
            ud = plsc.load_gather(urows_v, [rid, dcol])
            vd = plsc.load_gather(vrows_v, [rid, dcol])
            acc = acc + ud * vd
        out_v[pl.ds(g * L, L)] = acc
        return carry

    lax.fori_loop(0, BPW // L, body, 0)

    pltpu.sync_copy(out_v, out_hbm.at[pl.ds(wid * BPW, BPW)])


def kernel(x, U, V):
    x0 = x[:, 0].reshape(NW * NCHUNK, CHUNK)
    x1 = x[:, 1].reshape(NW * NCHUNK, CHUNK)
    return _mf_sc(x0, x1, U, V)
